# baseline (speedup 1.0000x reference)
import functools
from math import sqrt

import jax
import jax.numpy as jnp
from jax import lax
from jax.experimental import pallas as pl
from jax.experimental.pallas import tpu as pltpu
from jax.experimental.pallas import tpu_sc as plsc

NC = 2
NS = 16
NW = NC * NS
LANES = 16

SPLIT = 2
NBUF = 10
AHEAD = 6


@functools.lru_cache(maxsize=None)
def _build(V, D, NB, T):
    assert NB % NW == 0 and D % LANES == 0
    nb_w = NB // NW
    chb = nb_w // SPLIT
    n_ch = T * SPLIT
    assert chb % LANES == 0 and n_ch % NBUF == 0 and chb % 8 == 0
    scale = float(sqrt(D))

    mesh = plsc.VectorSubcoreMesh(core_axis_name="c", subcore_axis_name="s")

    @functools.partial(
        pl.kernel,
        mesh=mesh,
        out_type=jax.ShapeDtypeStruct((T, NB, D), jnp.float32),
        scratch_types=[
            pltpu.VMEM((T, nb_w), jnp.int32),
            pltpu.VMEM((NBUF, chb, D), jnp.float32),
        ]
        + [pltpu.SemaphoreType.DMA] * (2 * NBUF),
    )
    def emb_kernel(ids_hbm, table_hbm, out_hbm, idx_v, rows_v, *sems):
        gsem = sems[:NBUF]
        wsem = sems[NBUF:]
        wid = lax.axis_index("s") * NC + lax.axis_index("c")
        base = wid * nb_w

        pltpu.sync_copy(ids_hbm.at[:, pl.ds(base, nb_w)], idx_v)

        def gather(j, s):
            t, h = j // SPLIT, j % SPLIT
            return pltpu.make_async_copy(
                table_hbm.at[idx_v.at[t, pl.ds(h * chb, chb)]],
                rows_v.at[s], gsem[s])

        def write(j, s):
            t, h = j // SPLIT, j % SPLIT
            return pltpu.make_async_copy(
                rows_v.at[s], out_hbm.at[t, pl.ds(base + h * chb, chb)],
                wsem[s])

        def scale_slot(s):
            rref = rows_v.at[s]

            def row_body(r, carry):
                for c in range(D // LANES):
                    sl = pl.ds(c * LANES, LANES)
                    rref[r, sl] = rref[r, sl] * scale
                return carry

            lax.fori_loop(0, chb, row_body, 0, unroll=8)

        for j0 in range(AHEAD):
            gather(j0, j0).start()

        def outer(g, carry):
            for s in range(NBUF):
                j = g * NBUF + s
                sn = (s + AHEAD) % NBUF

                @pl.when(j + AHEAD < n_ch)
                def _refill():
                    @pl.when(j >= NBUF - AHEAD)
                    def _guard():
                        write(j - (NBUF - AHEAD), sn).wait()

                    gather(j + AHEAD, sn).start()

                gather(j, s).wait()
                scale_slot(s)
                write(j, s).start()

            return carry

        lax.fori_loop(0, n_ch // NBUF, outer, 0)

        for j0 in range(NBUF):
            jl = n_ch - NBUF + j0
            write(jl, jl % NBUF).wait()

    return emb_kernel


def kernel(input_ids, table):
    V, D = table.shape
    NB, T = input_ids.shape
    ids_t = input_ids.astype(jnp.int32).T
    out_tbd = _build(V, D, NB, T)(ids_t, table)
    return out_tbd.transpose(1, 0, 2)


# speedup vs baseline: 1.0044x; 1.0044x over previous
import functools
from math import sqrt

import jax
import jax.numpy as jnp
from jax import lax
from jax.experimental import pallas as pl
from jax.experimental.pallas import tpu as pltpu
from jax.experimental.pallas import tpu_sc as plsc

NC = 2
NS = 16
NW = NC * NS
LANES = 16

NBUF = 5
AHEAD = 4


@functools.lru_cache(maxsize=None)
def _build(V, D, NB, T):
    assert NB % NW == 0 and D % LANES == 0
    nb_w = NB // NW
    assert nb_w % LANES == 0 and T % NBUF == 0
    scale = float(sqrt(D))

    mesh = plsc.VectorSubcoreMesh(core_axis_name="c", subcore_axis_name="s")

    @functools.partial(
        pl.kernel,
        mesh=mesh,
        out_type=jax.ShapeDtypeStruct((T, NB, D), jnp.float32),
        scratch_types=[
            pltpu.VMEM((T, nb_w), jnp.int32),
            pltpu.VMEM((NBUF, nb_w, D), jnp.float32),
        ]
        + [pltpu.SemaphoreType.DMA] * (2 * NBUF),
    )
    def emb_kernel(ids_hbm, table_hbm, out_hbm, idx_v, rows_v, *sems):
        gsem = sems[:NBUF]
        wsem = sems[NBUF:]
        wid = lax.axis_index("s") * NC + lax.axis_index("c")
        base = wid * nb_w

        pltpu.sync_copy(ids_hbm.at[:, pl.ds(base, nb_w)], idx_v)

        def gather(t, s):
            return pltpu.make_async_copy(
                table_hbm.at[idx_v.at[t]], rows_v.at[s], gsem[s])

        def write(t, s):
            return pltpu.make_async_copy(
                rows_v.at[s], out_hbm.at[t, pl.ds(base, nb_w)], wsem[s])

        def scale_slot(s):
            rref = rows_v.at[s]

            def row_body(r, carry):
                for c in range(D // LANES):
                    sl = pl.ds(c * LANES, LANES)
                    rref[r, sl] = rref[r, sl] * scale
                return carry

            lax.fori_loop(0, nb_w, row_body, 0, unroll=8)

        for t0 in range(AHEAD):
            gather(t0, t0).start()

        def outer(g, carry):
            for s in range(NBUF):
                t = g * NBUF + s
                sn = (s + AHEAD) % NBUF

                @pl.when(t + AHEAD < T)
                def _refill():
                    @pl.when(t >= NBUF - AHEAD)
                    def _guard():
                        write(t - (NBUF - AHEAD), sn).wait()

                    gather(t + AHEAD, sn).start()

                gather(t, s).wait()
                scale_slot(s)
                write(t, s).start()

            return carry

        lax.fori_loop(0, T // NBUF, outer, 0)

        for t0 in range(NBUF):
            tl = T - NBUF + t0
            write(tl, tl % NBUF).wait()

    return emb_kernel


def kernel(input_ids, table):
    V, D = table.shape
    NB, T = input_ids.shape
    ids_t = input_ids.astype(jnp.int32).T
    out_tbd = _build(V, D, NB, T)(ids_t, table)
    return out_tbd.transpose(1, 0, 2)
